# baseline (speedup 1.0000x reference)
import functools

import jax
import jax.numpy as jnp
from jax.experimental import pallas as pl
from jax.experimental.pallas import tpu as pltpu

_N, _D, _H, _O = 10000, 128, 128, 128
_BX = 2000
_BI = 200


def _proj_kernel(x_ref, w_ref, b_ref, out_ref):
    h = jnp.dot(x_ref[...].astype(jnp.bfloat16), w_ref[...].astype(jnp.bfloat16),
                preferred_element_type=jnp.float32)
    out_ref[...] = (h + b_ref[...]).astype(jnp.bfloat16)


def _prop1_kernel(g_ref, h_ref, w2_ref, b2_ref, out_ref):
    g = g_ref[...].astype(jnp.bfloat16)
    p = jnp.dot(g, h_ref[...], preferred_element_type=jnp.float32)
    r = jnp.maximum(p, 0.0).astype(jnp.bfloat16)
    h2 = jnp.dot(r, w2_ref[...].astype(jnp.bfloat16),
                 preferred_element_type=jnp.float32) + b2_ref[...]
    out_ref[...] = h2.astype(jnp.bfloat16)


def _prop2_kernel(g_ref, h_ref, out_ref):
    g = g_ref[...].astype(jnp.bfloat16)
    out_ref[...] = jnp.dot(g, h_ref[...], preferred_element_type=jnp.float32)


def kernel(x, gcn, W1, b1, W2, b2):
    b1r = b1.reshape(1, _H)
    b2r = b2.reshape(1, _O)

    h1 = pl.pallas_call(
        _proj_kernel,
        grid=(_N // _BX,),
        in_specs=[
            pl.BlockSpec((_BX, _D), lambda i: (i, 0)),
            pl.BlockSpec((_D, _H), lambda i: (0, 0)),
            pl.BlockSpec((1, _H), lambda i: (0, 0)),
        ],
        out_specs=pl.BlockSpec((_BX, _H), lambda i: (i, 0)),
        out_shape=jax.ShapeDtypeStruct((_N, _H), jnp.bfloat16),
        compiler_params=pltpu.CompilerParams(
            dimension_semantics=("arbitrary",)),
    )(x, W1, b1r)

    h2 = pl.pallas_call(
        _prop1_kernel,
        grid=(_N // _BI,),
        in_specs=[
            pl.BlockSpec((_BI, _N), lambda i: (i, 0)),
            pl.BlockSpec((_N, _H), lambda i: (0, 0)),
            pl.BlockSpec((_H, _O), lambda i: (0, 0)),
            pl.BlockSpec((1, _O), lambda i: (0, 0)),
        ],
        out_specs=pl.BlockSpec((_BI, _O), lambda i: (i, 0)),
        out_shape=jax.ShapeDtypeStruct((_N, _O), jnp.bfloat16),
        compiler_params=pltpu.CompilerParams(
            dimension_semantics=("arbitrary",)),
    )(gcn, h1, W2, b2r)

    out = pl.pallas_call(
        _prop2_kernel,
        grid=(_N // _BI,),
        in_specs=[
            pl.BlockSpec((_BI, _N), lambda i: (i, 0)),
            pl.BlockSpec((_N, _O), lambda i: (0, 0)),
        ],
        out_specs=pl.BlockSpec((_BI, _O), lambda i: (i, 0)),
        out_shape=jax.ShapeDtypeStruct((_N, _O), jnp.float32),
        compiler_params=pltpu.CompilerParams(
            dimension_semantics=("arbitrary",)),
    )(gcn, h2)

    return out
